# baseline (speedup 1.0000x reference)
import functools

import jax
import jax.numpy as jnp
from jax import lax
from jax.experimental import pallas as pl
from jax.experimental.pallas import tpu as pltpu
from jax.experimental.pallas import tpu_sc as plsc

_EPS = 1e-12
_MARGIN = 1.0
_L = 16
_CHUNK = 64
_GROUP = 128


def _signed_unit(x):
    bits = lax.bitcast_convert_type(x, jnp.int32)
    one = jnp.int32(0x3F800000)
    sign_unit = lax.bitcast_convert_type(
        jnp.bitwise_or(jnp.bitwise_and(bits, jnp.int32(-0x80000000)), one),
        jnp.float32)
    return jnp.where(jnp.abs(x) >= _EPS, sign_unit, x * jnp.float32(1e12))


def _make_sc_kernel(nw, pb, d):
    mesh = plsc.VectorSubcoreMesh(core_axis_name="c", subcore_axis_name="s")
    info = plsc.get_sparse_core_info()
    nc = info.num_cores
    nch = pb // _CHUNK

    idx_t = pltpu.VMEM((pb,), jnp.int32)
    rows_t = pltpu.VMEM((2, _CHUNK, _GROUP), jnp.float32)

    @functools.partial(
        pl.kernel,
        mesh=mesh,
        out_type=jax.ShapeDtypeStruct((nw * _L,), jnp.float32),
        scratch_types=[idx_t] * 12 + [rows_t] * 6 + [
            pltpu.VMEM((_L,), jnp.float32),
            pltpu.SemaphoreType.DMA,
            pltpu.SemaphoreType.DMA,
            pltpu.SemaphoreType.DMA,
        ],
        compiler_params=pltpu.CompilerParams(needs_layout_passes=False),
    )
    def sc_kernel(phg, prg, ptg, nhg, nrg, ntg,
                  pho, pro, pto, nho, nro, nto,
                  ent, rel, out,
                  phgv, prgv, ptgv, nhgv, nrgv, ntgv,
                  phov, prov, ptov, nhov, nrov, ntov,
                  phr, prr, ptr, nhr, nrr, ntr,
                  accv, semi, semg0, semg1):
        wid = lax.axis_index("s") * nc + lax.axis_index("c")
        base = wid * pb

        idx_cps = []
        for src, dst in ((phg, phgv), (prg, prgv), (ptg, ptgv),
                         (nhg, nhgv), (nrg, nrgv), (ntg, ntgv),
                         (pho, phov), (pro, prov), (pto, ptov),
                         (nho, nhov), (nro, nrov), (nto, ntov)):
            idx_cps.append(
                pltpu.async_copy(src.at[pl.ds(base, pb)], dst, semi))
        for cp in idx_cps:
            cp.wait()

        gathers = [(ent, phgv, phr), (rel, prgv, prr), (ent, ptgv, ptr),
                   (ent, nhgv, nhr), (rel, nrgv, nrr), (ent, ntgv, ntr)]
        semg = (semg0, semg1)

        def fire(k):
            slot = k % 2
            return [pltpu.async_copy(
                        table.at[gv.at[pl.ds(k * _CHUNK, _CHUNK)]],
                        rows.at[slot], semg[slot])
                    for table, gv, rows in gathers]

        def compute(k, acc):
            slot = k % 2
            row_refs = (phr, prr, ptr, nhr, nrr, ntr)
            off_refs = (phov, prov, ptov, nhov, nrov, ntov)

            def blk(b, acc):
                obase = k * _CHUNK + b * _L
                ovecs = [ov[pl.ds(obase, _L)] for ov in off_refs]
                lanes = lax.iota(jnp.int32, _L)
                for i in range(_L):
                    rowv = jnp.full((_L,), b * _L + i, jnp.int32)
                    spl = jnp.full((_L,), i, jnp.int32)
                    halves = []
                    for rows, ov in zip(row_refs, ovecs):
                        col0 = jnp.take_along_axis(
                            ov, spl, axis=0, mode="promise_in_bounds") + lanes
                        halves.append(
                            (plsc.load_gather(rows.at[slot], [rowv, col0]),
                             plsc.load_gather(rows.at[slot],
                                              [rowv, col0 + _L])))
                    for c in (0, 1):
                        ph, pr, pt, nh, nr, nt = (h[c] for h in halves)
                        pos = jnp.abs(_signed_unit(ph) + pr - _signed_unit(pt))
                        neg = jnp.abs(_signed_unit(nh) + nr - _signed_unit(nt))
                        acc = acc + (pos - neg)
                return acc

            return lax.fori_loop(0, _CHUNK // _L, blk, acc)

        acc = jnp.zeros((_L,), jnp.float32)
        pending = fire(0)
        for k in range(1, nch + 1):
            if k <= nch - 1:
                nxt = fire(k)
            for cp in pending:
                cp.wait()
            if k <= nch - 1:
                pending = nxt
            acc = compute(k - 1, acc)

        accv[...] = acc
        pltpu.sync_copy(accv, out.at[pl.ds(wid * _L, _L)])

    return sc_kernel


def kernel(pos_exmpls, neg_exmpls, entity_emb, relation_emb):
    b, _ = pos_exmpls.shape
    _, d = entity_emb.shape
    info = plsc.get_sparse_core_info()
    nw = info.num_cores * info.num_subcores
    pb = b // nw
    gpr = _GROUP // d

    ent4 = entity_emb.reshape(-1, _GROUP)
    rel4 = relation_emb.reshape(-1, _GROUP)

    def grp(ex, c):
        return (ex[:, c].astype(jnp.int32) // gpr).reshape(-1)

    def off(ex, c):
        return ((ex[:, c].astype(jnp.int32) % gpr) * d).reshape(-1)

    sc = _make_sc_kernel(nw, pb, d)
    partials = sc(grp(pos_exmpls, 0), grp(pos_exmpls, 1), grp(pos_exmpls, 2),
                  grp(neg_exmpls, 0), grp(neg_exmpls, 1), grp(neg_exmpls, 2),
                  off(pos_exmpls, 0), off(pos_exmpls, 1), off(pos_exmpls, 2),
                  off(neg_exmpls, 0), off(neg_exmpls, 1), off(neg_exmpls, 2),
                  ent4, rel4)
    return jnp.maximum(jnp.sum(partials) + jnp.float32(_MARGIN),
                       jnp.float32(0.0))


# speedup vs baseline: 1.4848x; 1.4848x over previous
import functools

import jax
import jax.numpy as jnp
from jax import lax
from jax.experimental import pallas as pl
from jax.experimental.pallas import tpu as pltpu
from jax.experimental.pallas import tpu_sc as plsc

_EPS = 1e-12
_MARGIN = 1.0
_L = 16
_CHUNK = 32


def _signed_unit(x):
    bits = lax.bitcast_convert_type(x, jnp.int32)
    one = jnp.int32(0x3F800000)
    sign_unit = lax.bitcast_convert_type(
        jnp.bitwise_or(jnp.bitwise_and(bits, jnp.int32(-0x80000000)), one),
        jnp.float32)
    return jnp.where(jnp.abs(x) >= _EPS, sign_unit, x * jnp.float32(1e12))


def _make_sc_kernel(nw, pb, d):
    mesh = plsc.VectorSubcoreMesh(core_axis_name="c", subcore_axis_name="s")
    info = plsc.get_sparse_core_info()
    nc = info.num_cores
    nch = pb // _CHUNK

    idx_t = pltpu.VMEM((pb,), jnp.int32)
    rows_t = pltpu.VMEM((2, _CHUNK, d), jnp.float32)

    @functools.partial(
        pl.kernel,
        mesh=mesh,
        out_type=jax.ShapeDtypeStruct((nw * _L,), jnp.float32),
        scratch_types=[idx_t] * 6 + [rows_t] * 6 + [
            pltpu.VMEM((_L,), jnp.float32),
            pltpu.SemaphoreType.DMA,
            pltpu.SemaphoreType.DMA,
            pltpu.SemaphoreType.DMA,
        ],
        compiler_params=pltpu.CompilerParams(needs_layout_passes=False),
    )
    def sc_kernel(phi, pri, pti, nhi, nri, nti,
                  ent, rel, out,
                  phv, prv, ptv, nhv, nrv, ntv,
                  phr, prr, ptr, nhr, nrr, ntr,
                  accv, semi, semg0, semg1):
        wid = lax.axis_index("s") * nc + lax.axis_index("c")
        base = wid * pb

        idx_cps = []
        for src, dst in ((phi, phv), (pri, prv), (pti, ptv),
                         (nhi, nhv), (nri, nrv), (nti, ntv)):
            idx_cps.append(
                pltpu.async_copy(src.at[pl.ds(base, pb)], dst, semi))
        for cp in idx_cps:
            cp.wait()

        gathers = [(ent, phv, phr), (rel, prv, prr), (ent, ptv, ptr),
                   (ent, nhv, nhr), (rel, nrv, nrr), (ent, ntv, ntr)]
        semg = (semg0, semg1)

        def fire(k):
            slot = k % 2

            def issue(j, carry):
                for table, iv, rows in gathers:
                    iv16 = iv[pl.ds(k * _CHUNK + j * _L, _L)]
                    for l in range(_L):
                        pltpu.async_copy(
                            table.at[pl.ds(iv16[l], 1)],
                            rows.at[slot].at[pl.ds(j * _L + l, 1)],
                            semg[slot])
                return carry

            lax.fori_loop(0, _CHUNK // _L, issue, 0)

        def drain(k):
            slot = k % 2
            for table, iv, rows in gathers:
                pltpu.make_async_copy(
                    table.at[pl.ds(0, _CHUNK)], rows.at[slot],
                    semg[slot]).wait()

        def compute(k, acc):
            slot = k % 2
            fr = (phr, prr, ptr, nhr, nrr, ntr)

            def body(i, acc):
                for c in range(0, d, _L):
                    sl = pl.ds(c, _L)
                    ph, pr, pt, nh, nr, nt = (r[slot, i, sl] for r in fr)
                    pos = jnp.abs(_signed_unit(ph) + pr - _signed_unit(pt))
                    neg = jnp.abs(_signed_unit(nh) + nr - _signed_unit(nt))
                    acc = acc + (pos - neg)
                return acc

            return lax.fori_loop(0, _CHUNK, body, acc)

        acc = jnp.zeros((_L,), jnp.float32)
        fire(0)
        for k in range(1, nch + 1):
            if k <= nch - 1:
                fire(k)
            drain(k - 1)
            acc = compute(k - 1, acc)

        accv[...] = acc
        pltpu.sync_copy(accv, out.at[pl.ds(wid * _L, _L)])

    return sc_kernel


def kernel(pos_exmpls, neg_exmpls, entity_emb, relation_emb):
    b, _ = pos_exmpls.shape
    _, d = entity_emb.shape
    info = plsc.get_sparse_core_info()
    nw = info.num_cores * info.num_subcores
    pb = b // nw

    def col(ex, c):
        return ex[:, c].astype(jnp.int32).reshape(-1)

    sc = _make_sc_kernel(nw, pb, d)
    partials = sc(col(pos_exmpls, 0), col(pos_exmpls, 1), col(pos_exmpls, 2),
                  col(neg_exmpls, 0), col(neg_exmpls, 1), col(neg_exmpls, 2),
                  entity_emb, relation_emb)
    return jnp.maximum(jnp.sum(partials) + jnp.float32(_MARGIN),
                       jnp.float32(0.0))


# speedup vs baseline: 1.4975x; 1.0086x over previous
import functools

import jax
import jax.numpy as jnp
from jax import lax
from jax.experimental import pallas as pl
from jax.experimental.pallas import tpu as pltpu
from jax.experimental.pallas import tpu_sc as plsc

_EPS = 1e-12
_MARGIN = 1.0
_L = 16
_CHUNK = 32


def _signed_unit(x):
    bits = lax.bitcast_convert_type(x, jnp.int32)
    one = jnp.int32(0x3F800000)
    sign_unit = lax.bitcast_convert_type(
        jnp.bitwise_or(jnp.bitwise_and(bits, jnp.int32(-0x80000000)), one),
        jnp.float32)
    return jnp.where(jnp.abs(x) >= _EPS, sign_unit, x * jnp.float32(1e12))


def _make_sc_kernel(nw, pb, d):
    mesh = plsc.VectorSubcoreMesh(core_axis_name="c", subcore_axis_name="s")
    info = plsc.get_sparse_core_info()
    nc = info.num_cores
    nch = pb // _CHUNK

    idx_t = pltpu.VMEM((pb,), jnp.int32)
    rows_t = pltpu.VMEM((2, _CHUNK, d), jnp.float32)

    @functools.partial(
        pl.kernel,
        mesh=mesh,
        out_type=jax.ShapeDtypeStruct((nw * _L,), jnp.float32),
        scratch_types=[idx_t] * 6 + [rows_t] * 6 + [
            pltpu.VMEM((_L,), jnp.float32),
            pltpu.SemaphoreType.DMA,
            pltpu.SemaphoreType.DMA,
            pltpu.SemaphoreType.DMA,
        ],
        compiler_params=pltpu.CompilerParams(needs_layout_passes=False),
    )
    def sc_kernel(phi, pri, pti, nhi, nri, nti,
                  ent, rel, out,
                  phv, prv, ptv, nhv, nrv, ntv,
                  phr, prr, ptr, nhr, nrr, ntr,
                  accv, semi, semg0, semg1):
        wid = lax.axis_index("s") * nc + lax.axis_index("c")
        base = wid * pb

        idx_cps = []
        for src, dst in ((phi, phv), (pri, prv), (pti, ptv),
                         (nhi, nhv), (nri, nrv), (nti, ntv)):
            idx_cps.append(
                pltpu.async_copy(src.at[pl.ds(base, pb)], dst, semi))
        for cp in idx_cps:
            cp.wait()

        gathers = [(ent, phv, phr), (rel, prv, prr), (ent, ptv, ptr),
                   (ent, nhv, nhr), (rel, nrv, nrr), (ent, ntv, ntr)]
        semg = (semg0, semg1)

        def fire(k, slot):
            @plsc.parallel_loop(0, _CHUNK // _L)
            def issue(j):
                for table, iv, rows in gathers:
                    iv16 = iv[pl.ds(k * _CHUNK + j * _L, _L)]
                    for l in range(_L):
                        pltpu.async_copy(
                            table.at[pl.ds(iv16[l], 1)],
                            rows.at[slot].at[pl.ds(j * _L + l, 1)],
                            semg[slot])

        def drain(slot):
            for table, iv, rows in gathers:
                pltpu.make_async_copy(
                    table.at[pl.ds(0, _CHUNK)], rows.at[slot],
                    semg[slot]).wait()

        def compute(k, slot, acc):
            fr = (phr, prr, ptr, nhr, nrr, ntr)

            def body(i, acc):
                for c in range(0, d, _L):
                    sl = pl.ds(c, _L)
                    ph, pr, pt, nh, nr, nt = (r[slot, i, sl] for r in fr)
                    pos = jnp.abs(_signed_unit(ph) + pr - _signed_unit(pt))
                    neg = jnp.abs(_signed_unit(nh) + nr - _signed_unit(nt))
                    acc = acc + (pos - neg)
                return acc

            return lax.fori_loop(0, _CHUNK, body, acc)

        last = jnp.int32(nch - 1)

        def round_pair(kk, acc):
            k0 = kk * 2
            fire(jnp.minimum(k0 + 1, last), 1)
            drain(0)
            acc = compute(k0, 0, acc)
            fire(jnp.minimum(k0 + 2, last), 0)
            drain(1)
            return compute(k0 + 1, 1, acc)

        fire(jnp.int32(0), 0)
        acc = lax.fori_loop(0, nch // 2, round_pair,
                            jnp.zeros((_L,), jnp.float32))
        drain(0)

        accv[...] = acc
        pltpu.sync_copy(accv, out.at[pl.ds(wid * _L, _L)])

    return sc_kernel


def kernel(pos_exmpls, neg_exmpls, entity_emb, relation_emb):
    b, _ = pos_exmpls.shape
    _, d = entity_emb.shape
    info = plsc.get_sparse_core_info()
    nw = info.num_cores * info.num_subcores
    pb = b // nw

    def col(ex, c):
        return ex[:, c].astype(jnp.int32).reshape(-1)

    sc = _make_sc_kernel(nw, pb, d)
    partials = sc(col(pos_exmpls, 0), col(pos_exmpls, 1), col(pos_exmpls, 2),
                  col(neg_exmpls, 0), col(neg_exmpls, 1), col(neg_exmpls, 2),
                  entity_emb, relation_emb)
    return jnp.maximum(jnp.sum(partials) + jnp.float32(_MARGIN),
                       jnp.float32(0.0))
